# baseline (speedup 1.0000x reference)
import jax
import jax.numpy as jnp
from jax.experimental import pallas as pl
from jax.experimental.pallas import tpu as pltpu

KEEP = 0.5
N_NODES = 4096
DIM = 512
BLK = 512
GRID = N_NODES // BLK
N_LAYERS = 4


def _mm(a, b):
    return jax.lax.dot_general(
        a, b,
        (((1,), (0,)), ((), ())),
        preferred_element_type=jnp.float32,
    )


def _fused_kernel(f_ref, adj_ref, w_ref, out_ref, h_ref, t_ref):
    l = pl.program_id(0)
    g = pl.program_id(1)
    cur = jax.lax.rem(l, 2)
    nxt = 1 - cur

    @pl.when((l == 0) & (g == 0))
    def _():
        t_ref[0] = _mm(f_ref[...], w_ref[0])

    rows = pl.ds(g * BLK, BLK)
    h_in = jnp.where(l == 0, f_ref[rows, :], h_ref[rows, :])
    prop = _mm(adj_ref[0], t_ref[cur])
    new_h = jnp.maximum(KEEP * h_in + (1.0 - KEEP) * prop, 0.0)
    h_ref[rows, :] = new_h

    @pl.when(l < N_LAYERS - 1)
    def _():
        t_ref[nxt, rows, :] = _mm(new_h, w_ref[l + 1])

    @pl.when(l == N_LAYERS - 1)
    def _():
        out_ref[...] = new_h


@jax.jit
def kernel(features, adj_lst, W):
    out = pl.pallas_call(
        _fused_kernel,
        grid=(N_LAYERS, GRID),
        in_specs=[
            pl.BlockSpec((N_NODES, DIM), lambda l, g: (0, 0)),
            pl.BlockSpec((1, BLK, N_NODES), lambda l, g: (l, g, 0)),
            pl.BlockSpec((N_LAYERS, DIM, DIM), lambda l, g: (0, 0, 0)),
        ],
        out_specs=pl.BlockSpec(
            (BLK, DIM),
            lambda l, g: (jnp.where(l == N_LAYERS - 1, g, 0), 0),
        ),
        out_shape=jax.ShapeDtypeStruct((N_NODES, DIM), jnp.float32),
        scratch_shapes=[
            pltpu.VMEM((N_NODES, DIM), jnp.float32),
            pltpu.VMEM((2, N_NODES, DIM), jnp.float32),
        ],
        compiler_params=pltpu.CompilerParams(
            dimension_semantics=("arbitrary", "arbitrary"),
            vmem_limit_bytes=64 * 1024 * 1024,
        ),
    )(features, adj_lst, W)
    return out
